# Initial kernel scaffold; baseline (speedup 1.0000x reference)
import jax
import jax.numpy as jnp
from jax.experimental import pallas as pl


def kernel(x, item_starts, batch_idx, batch_len, table, W0, b0, Wh, bh):
    raise NotImplementedError("write your pallas kernel here")


# speedup vs baseline: 2.8798x; 2.8798x over previous
import functools

import jax
import jax.numpy as jnp
from jax import lax
from jax.experimental import pallas as pl
from jax.experimental.pallas import tpu as pltpu
from jax.experimental.pallas import tpu_sc as plsc

EDIM = 64
N_AGENTS = 50000
N_BATCH = 1000
NB_PAD = 1008

_NC = 2
_NS = 16
_NW = _NC * _NS

_N_PAD = 50176
_ROWS_PER_W = _N_PAD // _NW
_CHUNK = 112
_NCHUNK = _ROWS_PER_W // _CHUNK
_NGRP = _CHUNK // 16

_TC_BLK = 512
_TC_GRID = _N_PAD // _TC_BLK


def _sc_mesh():
    return plsc.VectorSubcoreMesh(core_axis_name="c", subcore_axis_name="s")


def _stage1(x_pad, table, bidx_pad, bl_pad, zeros_m):
    @functools.partial(
        pl.kernel,
        out_type=(
            jax.ShapeDtypeStruct((_N_PAD, EDIM), jnp.float32),
            jax.ShapeDtypeStruct((_NC, NB_PAD, EDIM), jnp.float32),
            jax.ShapeDtypeStruct((_N_PAD,), jnp.float32),
        ),
        mesh=_sc_mesh(),
        compiler_params=pltpu.CompilerParams(use_tc_tiling_on_sc=False, needs_layout_passes=False),
        scratch_types=[
            pltpu.VMEM((_NCHUNK, _CHUNK), jnp.int32),
            pltpu.VMEM((_NCHUNK, _CHUNK), jnp.int32),
            pltpu.VMEM((2, _CHUNK, EDIM), jnp.float32),
            pltpu.VMEM((NB_PAD,), jnp.float32),
            pltpu.VMEM((_CHUNK,), jnp.float32),
            pltpu.VMEM_SHARED((NB_PAD, EDIM), jnp.float32),
            pltpu.SemaphoreType.DMA,
            pltpu.SemaphoreType.DMA,
        ],
    )
    def k(x_hbm, table_hbm, bidx_hbm, bl_hbm, zeros_hbm,
          emb_hbm, mpart_hbm, dexp_hbm,
          idx_v, bidx_v, rows_v, bl_v, dexp_v, m_sh, sem0, sem1):
        cid = lax.axis_index("c")
        sid = lax.axis_index("s")
        wid = sid * _NC + cid
        base = wid * _ROWS_PER_W
        for c in range(_NCHUNK):
            off = base + c * _CHUNK
            pltpu.sync_copy(x_hbm.at[pl.ds(off, _CHUNK)], idx_v.at[c])
            pltpu.sync_copy(bidx_hbm.at[pl.ds(off, _CHUNK)], bidx_v.at[c])
        pltpu.sync_copy(bl_hbm, bl_v)

        @pl.when(sid == 0)
        def _():
            pltpu.sync_copy(zeros_hbm, m_sh)

        plsc.subcore_barrier()

        sems = (sem0, sem1)
        cps = [None, None]
        cps[0] = pltpu.async_copy(table_hbm.at[idx_v.at[0]], rows_v.at[0],
                                  sems[0])
        for c in range(_NCHUNK):
            nxt = c + 1
            if nxt < _NCHUNK:
                cps[nxt % 2] = pltpu.async_copy(
                    table_hbm.at[idx_v.at[nxt]], rows_v.at[nxt % 2],
                    sems[nxt % 2])
            cps[c % 2].wait()
            buf = rows_v.at[c % 2]
            pltpu.sync_copy(buf, emb_hbm.at[pl.ds(base + c * _CHUNK, _CHUNK)])
            pltpu.sync_copy(buf, m_sh.at[bidx_v.at[c]], add=True)
            for grp in range(_NGRP):
                gv = bidx_v[c, pl.ds(grp * 16, 16)]
                blv = plsc.load_gather(bl_v, [gv])
                dexp_v[pl.ds(grp * 16, 16)] = 1.0 / (blv - 0.99999)
            pltpu.sync_copy(dexp_v,
                            dexp_hbm.at[pl.ds(base + c * _CHUNK, _CHUNK)])

        plsc.subcore_barrier()

        @pl.when(sid == 0)
        def _():
            pltpu.sync_copy(m_sh, mpart_hbm.at[cid])

    return k(x_pad, table, bidx_pad, bl_pad, zeros_m)


def _stage_tc(emb, dexp2d, m_part, W0, b02d, bl2d):
    def body(emb_ref, dexp_ref, mpart_ref, w0_ref, b0_ref, bl_ref,
             v_ref, rp_ref):
        a_t = w0_ref[:, :EDIM]
        b_t = w0_ref[:, EDIM:]
        eb = emb_ref[...]
        dn = (((1,), (1,)), ((), ()))
        h = lax.dot_general(eb, a_t, dn, preferred_element_type=jnp.float32)
        q = lax.dot_general(eb, b_t, dn, preferred_element_type=jnp.float32)
        de = dexp_ref[...]
        v_ref[...] = h + b0_ref[...] - q * de

        @pl.when(pl.program_id(0) == 0)
        def _():
            m = mpart_ref[0] + mpart_ref[1]
            r = lax.dot_general(m, b_t, dn, preferred_element_type=jnp.float32)
            rp_ref[...] = r * (1.0 / (bl_ref[...] - 0.99999))

    return pl.pallas_call(
        body,
        grid=(_TC_GRID,),
        in_specs=[
            pl.BlockSpec((_TC_BLK, EDIM), lambda b: (b, 0)),
            pl.BlockSpec((_TC_BLK, 1), lambda b: (b, 0)),
            pl.BlockSpec((_NC, NB_PAD, EDIM), lambda b: (0, 0, 0)),
            pl.BlockSpec((EDIM, 2 * EDIM), lambda b: (0, 0)),
            pl.BlockSpec((1, EDIM), lambda b: (0, 0)),
            pl.BlockSpec((NB_PAD, 1), lambda b: (0, 0)),
        ],
        out_specs=[
            pl.BlockSpec((_TC_BLK, EDIM), lambda b: (b, 0)),
            pl.BlockSpec((NB_PAD, EDIM), lambda b: (0, 0)),
        ],
        out_shape=[
            jax.ShapeDtypeStruct((_N_PAD, EDIM), jnp.float32),
            jax.ShapeDtypeStruct((NB_PAD, EDIM), jnp.float32),
        ],
    )(emb, dexp2d, m_part, W0, b02d, bl2d)


_NB_CHUNK = 112
_NB_NCH = NB_PAD // _NB_CHUNK


def _stage2(v_arr, rp, bidx_pad, wh_pad, zeros_out, ident):
    @functools.partial(
        pl.kernel,
        out_type=jax.ShapeDtypeStruct((_NC, NB_PAD, 16), jnp.float32),
        mesh=_sc_mesh(),
        compiler_params=pltpu.CompilerParams(use_tc_tiling_on_sc=False, needs_layout_passes=False),
        scratch_types=[
            pltpu.VMEM((_NCHUNK, _CHUNK), jnp.int32),
            pltpu.VMEM((2, _CHUNK, EDIM), jnp.float32),
            pltpu.VMEM((NB_PAD, EDIM), jnp.float32),
            pltpu.VMEM((EDIM,), jnp.float32),
            pltpu.VMEM((NB_PAD, 16), jnp.float32),
            pltpu.VMEM((_NB_NCH, _NB_CHUNK), jnp.int32),
            pltpu.VMEM_SHARED((NB_PAD, 16), jnp.float32),
            pltpu.SemaphoreType.DMA((2,)),
        ],
    )
    def k(v_hbm, rp_hbm, bidx_hbm, wh_hbm, zeros_hbm, ident_hbm, out_hbm,
          bidx_v, v_v, rp_v, wh_v, acc_v, ident_v, out_sh, sem):
        cid = lax.axis_index("c")
        sid = lax.axis_index("s")
        wid = sid * _NC + cid
        base = wid * _ROWS_PER_W
        for c in range(_NCHUNK):
            off = base + c * _CHUNK
            pltpu.sync_copy(bidx_hbm.at[pl.ds(off, _CHUNK)], bidx_v.at[c])
        pltpu.sync_copy(rp_hbm, rp_v)
        pltpu.sync_copy(wh_hbm, wh_v)
        pltpu.sync_copy(ident_hbm, ident_v)
        whs = [wh_v[pl.ds(qq * 16, 16)] for qq in range(4)]
        zero16 = jnp.zeros((16,), jnp.float32)

        def zinit(i, _):
            acc_v[i, :] = zero16
            return 0
        lax.fori_loop(0, NB_PAD, zinit, 0)

        @pl.when(sid == 0)
        def _():
            pltpu.sync_copy(zeros_hbm, out_sh)

        plsc.subcore_barrier()

        pltpu.make_async_copy(
            v_hbm.at[pl.ds(base, _CHUNK)], v_v.at[0], sem.at[0]).start()

        def do_chunk(c, _):
            nxt = c + 1
            nbuf = lax.rem(nxt, 2)
            cbuf = lax.rem(c, 2)

            @pl.when(nxt < _NCHUNK)
            def _():
                pltpu.make_async_copy(
                    v_hbm.at[pl.ds(pl.multiple_of(base + nxt * _CHUNK, 16),
                                   _CHUNK)],
                    v_v.at[nbuf], sem.at[nbuf]).start()

            pltpu.make_async_copy(
                v_hbm.at[pl.ds(base, _CHUNK)], v_v.at[cbuf],
                sem.at[cbuf]).wait()

            def do_grp(grp, _):
                gv = bidx_v[c, pl.ds(grp * 16, 16)]
                row0 = grp * 16
                for j in range(16):
                    g = gv[j]
                    p = zero16
                    for q in range(4):
                        vv = (v_v[cbuf, row0 + j, pl.ds(q * 16, 16)]
                              + rp_v[g, pl.ds(q * 16, 16)])
                        vv = jnp.maximum(vv, 0.0)
                        p = p + vv * whs[q]
                    plsc.addupdate(acc_v.at[g], p)
                return 0

            lax.fori_loop(0, _NGRP, do_grp, 0)
            return 0

        lax.fori_loop(0, _NCHUNK, do_chunk, 0)

        for cc in range(_NB_NCH):
            pltpu.sync_copy(acc_v.at[pl.ds(cc * _NB_CHUNK, _NB_CHUNK)],
                            out_sh.at[ident_v.at[cc]], add=True)
        plsc.subcore_barrier()

        @pl.when(sid == 0)
        def _():
            pltpu.sync_copy(out_sh, out_hbm.at[cid])

    return k(v_arr, rp, bidx_pad, wh_pad, zeros_out, ident)


def kernel(x, item_starts, batch_idx, batch_len, table, W0, b0, Wh, bh):
    del item_starts
    pad = _N_PAD - N_AGENTS
    x_pad = jnp.concatenate([x, jnp.zeros((pad,), dtype=x.dtype)])
    bidx_pad = jnp.concatenate(
        [batch_idx, jnp.full((pad,), N_BATCH, dtype=batch_idx.dtype)])
    bl_pad = jnp.concatenate(
        [batch_len, jnp.ones((NB_PAD - N_BATCH,), dtype=batch_len.dtype)])
    zeros_m = jnp.zeros((NB_PAD, EDIM), jnp.float32)
    zeros_out = jnp.zeros((NB_PAD, 16), jnp.float32)

    emb, m_part, dexp = _stage1(x_pad, table, bidx_pad, bl_pad, zeros_m)
    v_arr, rp = _stage_tc(emb, dexp[:, None],
                          m_part, W0, jnp.reshape(b0, (1, EDIM)),
                          jnp.reshape(bl_pad, (NB_PAD, 1)))
    ident = jnp.arange(NB_PAD, dtype=jnp.int32).reshape(_NB_NCH, _NB_CHUNK)
    out_part = _stage2(v_arr, rp, bidx_pad, Wh[0], zeros_out, ident)
    out = jnp.sum(out_part, axis=(0, 2))[:N_BATCH, None] + bh
    return out


# speedup vs baseline: 3.1482x; 1.0932x over previous
import functools

import jax
import jax.numpy as jnp
from jax import lax
from jax.experimental import pallas as pl
from jax.experimental.pallas import tpu as pltpu
from jax.experimental.pallas import tpu_sc as plsc

EDIM = 64
N_AGENTS = 50000
N_BATCH = 1000
NB_PAD = 1008

_NC = 2
_NS = 16
_NW = _NC * _NS

_N_PAD = 50176
_ROWS_PER_W = _N_PAD // _NW
_CHUNK = 112
_NCHUNK = _ROWS_PER_W // _CHUNK
_NGRP = _CHUNK // 16

_TC_BLK = 512
_TC_GRID = _N_PAD // _TC_BLK


def _sc_mesh():
    return plsc.VectorSubcoreMesh(core_axis_name="c", subcore_axis_name="s")


_WIDE = 128


def _stage1(x_pad, table, bidx_pad, bl_pad, zeros_m):
    @functools.partial(
        pl.kernel,
        out_type=(
            jax.ShapeDtypeStruct((_N_PAD, _WIDE), jnp.float32),
            jax.ShapeDtypeStruct((_NC, NB_PAD, _WIDE), jnp.float32),
            jax.ShapeDtypeStruct((_N_PAD,), jnp.float32),
        ),
        mesh=_sc_mesh(),
        compiler_params=pltpu.CompilerParams(use_tc_tiling_on_sc=True, needs_layout_passes=False),
        scratch_types=[
            pltpu.VMEM((_NCHUNK, _CHUNK), jnp.int32),
            pltpu.VMEM((_NCHUNK, _CHUNK), jnp.int32),
            pltpu.VMEM((2, _CHUNK, _WIDE), jnp.float32),
            pltpu.VMEM((NB_PAD,), jnp.float32),
            pltpu.VMEM((_CHUNK,), jnp.float32),
            pltpu.VMEM_SHARED((NB_PAD, _WIDE), jnp.float32),
            pltpu.SemaphoreType.DMA,
            pltpu.SemaphoreType.DMA,
        ],
    )
    def k(x_hbm, table_hbm, bidx_hbm, bl_hbm, zeros_hbm,
          emb_hbm, mpart_hbm, dexp_hbm,
          idx_v, bidx_v, rows_v, bl_v, dexp_v, m_sh, sem0, sem1):
        cid = lax.axis_index("c")
        sid = lax.axis_index("s")
        wid = sid * _NC + cid
        base = wid * _ROWS_PER_W
        for c in range(_NCHUNK):
            off = base + c * _CHUNK
            pltpu.sync_copy(x_hbm.at[pl.ds(off, _CHUNK)], idx_v.at[c])
            pltpu.sync_copy(bidx_hbm.at[pl.ds(off, _CHUNK)], bidx_v.at[c])
        pltpu.sync_copy(bl_hbm, bl_v)

        @pl.when(sid == 0)
        def _():
            pltpu.sync_copy(zeros_hbm, m_sh)

        plsc.subcore_barrier()

        sems = (sem0, sem1)
        cps = [None, None]
        cps[0] = pltpu.async_copy(table_hbm.at[idx_v.at[0]], rows_v.at[0],
                                  sems[0])
        for c in range(_NCHUNK):
            nxt = c + 1
            if nxt < _NCHUNK:
                cps[nxt % 2] = pltpu.async_copy(
                    table_hbm.at[idx_v.at[nxt]], rows_v.at[nxt % 2],
                    sems[nxt % 2])
            cps[c % 2].wait()
            buf = rows_v.at[c % 2]
            pltpu.sync_copy(buf, emb_hbm.at[pl.ds(base + c * _CHUNK, _CHUNK)])
            pltpu.sync_copy(buf, m_sh.at[bidx_v.at[c]], add=True)
            for grp in range(_NGRP):
                gv = bidx_v[c, pl.ds(grp * 16, 16)]
                blv = plsc.load_gather(bl_v, [gv])
                dexp_v[pl.ds(grp * 16, 16)] = 1.0 / (blv - 0.99999)
            pltpu.sync_copy(dexp_v,
                            dexp_hbm.at[pl.ds(base + c * _CHUNK, _CHUNK)])

        plsc.subcore_barrier()

        @pl.when(sid == 0)
        def _():
            pltpu.sync_copy(m_sh, mpart_hbm.at[cid])

    return k(x_pad, table, bidx_pad, bl_pad, zeros_m)


def _stage_tc(emb, dexp2d, m_part, W0, b02d, bl2d):
    def body(emb_ref, dexp_ref, mpart_ref, w0_ref, b0_ref, bl_ref,
             v_ref, rp_ref):
        a_t = w0_ref[:, :EDIM]
        b_t = w0_ref[:, EDIM:]
        eb = emb_ref[:, :EDIM]
        dn = (((1,), (1,)), ((), ()))
        h = lax.dot_general(eb, a_t, dn, preferred_element_type=jnp.float32)
        q = lax.dot_general(eb, b_t, dn, preferred_element_type=jnp.float32)
        de = dexp_ref[...]
        v_ref[...] = h + b0_ref[...] - q * de

        @pl.when(pl.program_id(0) == 0)
        def _():
            m = (mpart_ref[0] + mpart_ref[1])[:, :EDIM]
            r = lax.dot_general(m, b_t, dn, preferred_element_type=jnp.float32)
            rp_ref[...] = r * (1.0 / (bl_ref[...] - 0.99999))

    return pl.pallas_call(
        body,
        grid=(_TC_GRID,),
        in_specs=[
            pl.BlockSpec((_TC_BLK, _WIDE), lambda b: (b, 0)),
            pl.BlockSpec((_TC_BLK, 1), lambda b: (b, 0)),
            pl.BlockSpec((_NC, NB_PAD, _WIDE), lambda b: (0, 0, 0)),
            pl.BlockSpec((EDIM, 2 * EDIM), lambda b: (0, 0)),
            pl.BlockSpec((1, EDIM), lambda b: (0, 0)),
            pl.BlockSpec((NB_PAD, 1), lambda b: (0, 0)),
        ],
        out_specs=[
            pl.BlockSpec((_TC_BLK, EDIM), lambda b: (b, 0)),
            pl.BlockSpec((NB_PAD, EDIM), lambda b: (0, 0)),
        ],
        out_shape=[
            jax.ShapeDtypeStruct((_N_PAD, EDIM), jnp.float32),
            jax.ShapeDtypeStruct((NB_PAD, EDIM), jnp.float32),
        ],
    )(emb, dexp2d, m_part, W0, b02d, bl2d)


_NB_CHUNK = 112
_NB_NCH = NB_PAD // _NB_CHUNK


def _stage2(v_arr, rp, bidx_pad, wh_pad, zeros_out, ident):
    @functools.partial(
        pl.kernel,
        out_type=jax.ShapeDtypeStruct((_NC, NB_PAD, 16), jnp.float32),
        mesh=_sc_mesh(),
        compiler_params=pltpu.CompilerParams(use_tc_tiling_on_sc=False, needs_layout_passes=False),
        scratch_types=[
            pltpu.VMEM((_NCHUNK, _CHUNK), jnp.int32),
            pltpu.VMEM((2, _CHUNK, EDIM), jnp.float32),
            pltpu.VMEM((NB_PAD, EDIM), jnp.float32),
            pltpu.VMEM((EDIM,), jnp.float32),
            pltpu.VMEM((NB_PAD, 16), jnp.float32),
            pltpu.VMEM((_NB_NCH, _NB_CHUNK), jnp.int32),
            pltpu.VMEM_SHARED((NB_PAD, 16), jnp.float32),
            pltpu.SemaphoreType.DMA((2,)),
        ],
    )
    def k(v_hbm, rp_hbm, bidx_hbm, wh_hbm, zeros_hbm, ident_hbm, out_hbm,
          bidx_v, v_v, rp_v, wh_v, acc_v, ident_v, out_sh, sem):
        cid = lax.axis_index("c")
        sid = lax.axis_index("s")
        wid = sid * _NC + cid
        base = wid * _ROWS_PER_W
        for c in range(_NCHUNK):
            off = base + c * _CHUNK
            pltpu.sync_copy(bidx_hbm.at[pl.ds(off, _CHUNK)], bidx_v.at[c])
        pltpu.sync_copy(rp_hbm, rp_v)
        pltpu.sync_copy(wh_hbm, wh_v)
        pltpu.sync_copy(ident_hbm, ident_v)
        whs = [wh_v[pl.ds(qq * 16, 16)] for qq in range(4)]
        zero16 = jnp.zeros((16,), jnp.float32)

        def zinit(i, _):
            acc_v[i, :] = zero16
            return 0
        lax.fori_loop(0, NB_PAD, zinit, 0)

        @pl.when(sid == 0)
        def _():
            pltpu.sync_copy(zeros_hbm, out_sh)

        plsc.subcore_barrier()

        pltpu.make_async_copy(
            v_hbm.at[pl.ds(base, _CHUNK)], v_v.at[0], sem.at[0]).start()

        def do_chunk(c, _):
            nxt = c + 1
            nbuf = lax.rem(nxt, 2)
            cbuf = lax.rem(c, 2)

            @pl.when(nxt < _NCHUNK)
            def _():
                pltpu.make_async_copy(
                    v_hbm.at[pl.ds(pl.multiple_of(base + nxt * _CHUNK, 16),
                                   _CHUNK)],
                    v_v.at[nbuf], sem.at[nbuf]).start()

            pltpu.make_async_copy(
                v_hbm.at[pl.ds(base, _CHUNK)], v_v.at[cbuf],
                sem.at[cbuf]).wait()

            def do_grp(grp, _):
                gv = bidx_v[c, pl.ds(grp * 16, 16)]
                row0 = grp * 16
                for j in range(16):
                    g = gv[j]
                    p = zero16
                    for q in range(4):
                        vv = (v_v[cbuf, row0 + j, pl.ds(q * 16, 16)]
                              + rp_v[g, pl.ds(q * 16, 16)])
                        vv = jnp.maximum(vv, 0.0)
                        p = p + vv * whs[q]
                    plsc.addupdate(acc_v.at[g], p)
                return 0

            lax.fori_loop(0, _NGRP, do_grp, 0)
            return 0

        lax.fori_loop(0, _NCHUNK, do_chunk, 0)

        for cc in range(_NB_NCH):
            pltpu.sync_copy(acc_v.at[pl.ds(cc * _NB_CHUNK, _NB_CHUNK)],
                            out_sh.at[ident_v.at[cc]], add=True)
        plsc.subcore_barrier()

        @pl.when(sid == 0)
        def _():
            pltpu.sync_copy(out_sh, out_hbm.at[cid])

    return k(v_arr, rp, bidx_pad, wh_pad, zeros_out, ident)


def kernel(x, item_starts, batch_idx, batch_len, table, W0, b0, Wh, bh):
    del item_starts
    pad = _N_PAD - N_AGENTS
    x_pad = jnp.concatenate([x, jnp.zeros((pad,), dtype=x.dtype)])
    bidx_pad = jnp.concatenate(
        [batch_idx, jnp.full((pad,), N_BATCH, dtype=batch_idx.dtype)])
    bl_pad = jnp.concatenate(
        [batch_len, jnp.ones((NB_PAD - N_BATCH,), dtype=batch_len.dtype)])
    zeros_m = jnp.zeros((NB_PAD, _WIDE), jnp.float32)
    zeros_out = jnp.zeros((NB_PAD, 16), jnp.float32)

    table_w = jnp.pad(table, ((0, 0), (0, _WIDE - EDIM)))
    emb, m_part, dexp = _stage1(x_pad, table_w, bidx_pad, bl_pad, zeros_m)
    v_arr, rp = _stage_tc(emb, dexp[:, None],
                          m_part, W0, jnp.reshape(b0, (1, EDIM)),
                          jnp.reshape(bl_pad, (NB_PAD, 1)))
    ident = jnp.arange(NB_PAD, dtype=jnp.int32).reshape(_NB_NCH, _NB_CHUNK)
    out_part = _stage2(v_arr, rp, bidx_pad, Wh[0], zeros_out, ident)
    out = jnp.sum(out_part, axis=(0, 2))[:N_BATCH, None] + bh
    return out
